# Initial kernel scaffold; baseline (speedup 1.0000x reference)
import jax
import jax.numpy as jnp
from jax.experimental import pallas as pl


def kernel(node_fts, edge_index, W_neigh1, W_self1, b1, W_neigh2, W_self2, b2):
    raise NotImplementedError("write your pallas kernel here")


# speedup vs baseline: 5.0868x; 5.0868x over previous
import jax
import jax.numpy as jnp
from jax import lax
from jax.experimental import pallas as pl
from jax.experimental.pallas import tpu as pltpu
from jax.experimental.pallas import tpu_sc as plsc

N = 10000
E = 320000
D = 128

NC = 2
NS = 16
NW = NC * NS
L = 16

B = 128
KB = -(-E // (NW * B))
E_PAD = NW * KB * B
N_PAD = 10240
RPT = N_PAD // NS
ZR = 16

_mesh = plsc.VectorSubcoreMesh(
    core_axis_name="c", subcore_axis_name="s", num_cores=NC, num_subcores=NS)


def _sc_body(with_deg, *refs):
    if with_deg:
        (x_hbm, src_hbm, dst_hbm, agg_hbm, deg_hbm,
         src_v, dst_v, rows_v, zbuf_v, zvec_v, ones_v, acc_sh, dacc_sh, sem) = refs
    else:
        (x_hbm, src_hbm, dst_hbm, agg_hbm,
         src_v, dst_v, rows_v, zbuf_v, acc_sh, sem) = refs

    cid = lax.axis_index("c")
    sid = lax.axis_index("s")
    wid = sid * NC + cid
    r0 = sid * RPT

    zv = jnp.zeros((L,), jnp.float32)
    for i in range(ZR):
        for j in range(D // L):
            zbuf_v[i, pl.ds(j * L, L)] = zv
    if with_deg:
        for j in range(RPT // L):
            zvec_v[pl.ds(j * L, L)] = zv
        ov = jnp.ones((L,), jnp.float32)
        for j in range(B // L):
            ones_v[pl.ds(j * L, L)] = ov

    def _zero(k, carry):
        pltpu.sync_copy(zbuf_v, acc_sh.at[pl.ds(r0 + k * ZR, ZR)])
        return carry
    lax.fori_loop(0, RPT // ZR, _zero, 0)
    if with_deg:
        pltpu.sync_copy(zvec_v, dacc_sh.at[pl.ds(r0, RPT)])

    pltpu.sync_copy(src_hbm.at[wid], src_v)
    pltpu.sync_copy(dst_hbm.at[wid], dst_v)

    plsc.subcore_barrier()

    def _edge_block(j, carry):
        pltpu.async_copy(x_hbm.at[src_v.at[j]], rows_v, sem).wait()
        pltpu.sync_copy(rows_v, acc_sh.at[dst_v.at[j]], add=True)
        if with_deg:
            pltpu.sync_copy(ones_v, dacc_sh.at[dst_v.at[j]], add=True)
        return carry
    lax.fori_loop(0, KB, _edge_block, 0)

    plsc.subcore_barrier()

    pltpu.sync_copy(acc_sh.at[pl.ds(r0, RPT)],
                    agg_hbm.at[cid].at[pl.ds(r0, RPT)])
    if with_deg:
        pltpu.sync_copy(dacc_sh.at[pl.ds(r0, RPT)],
                        deg_hbm.at[cid].at[pl.ds(r0, RPT)])


def _make_sc(with_deg):
    out_type = [jax.ShapeDtypeStruct((NC, N_PAD, D), jnp.float32)]
    scratch = [
        pltpu.VMEM((KB, B), jnp.int32),
        pltpu.VMEM((KB, B), jnp.int32),
        pltpu.VMEM((B, D), jnp.float32),
        pltpu.VMEM((ZR, D), jnp.float32),
    ]
    if with_deg:
        out_type.append(jax.ShapeDtypeStruct((NC, N_PAD), jnp.float32))
        scratch += [
            pltpu.VMEM((RPT,), jnp.float32),
            pltpu.VMEM((B,), jnp.float32),
        ]
    scratch.append(pltpu.VMEM_SHARED((N_PAD, D), jnp.float32))
    if with_deg:
        scratch.append(pltpu.VMEM_SHARED((N_PAD,), jnp.float32))
    scratch.append(pltpu.SemaphoreType.DMA)

    def body(*refs):
        _sc_body(with_deg, *refs)
    return pl.kernel(body, out_type=out_type, mesh=_mesh, scratch_types=scratch)


_sc_agg_deg = _make_sc(True)
_sc_agg = _make_sc(False)

BR = 512


def _tc_body(agg_ref, deg_ref, x_ref, wn_ref, ws_ref, b_ref, o_ref):
    agg = agg_ref[0] + agg_ref[1]
    deg = deg_ref[0] + deg_ref[1]
    recip = 1.0 / jnp.maximum(deg, 1.0)
    m = jnp.dot(agg, wn_ref[...], preferred_element_type=jnp.float32)
    h = (m * recip
         + jnp.dot(x_ref[...], ws_ref[...], preferred_element_type=jnp.float32)
         + b_ref[...])
    o_ref[...] = jnp.maximum(h, 0.0)


def _tc_layer(agg, deg3, x, wn, ws, b):
    nb = N_PAD // BR
    return pl.pallas_call(
        _tc_body,
        grid=(nb,),
        in_specs=[
            pl.BlockSpec((NC, BR, D), lambda i: (0, i, 0)),
            pl.BlockSpec((NC, BR, 1), lambda i: (0, i, 0)),
            pl.BlockSpec((BR, D), lambda i: (i, 0)),
            pl.BlockSpec((D, D), lambda i: (0, 0)),
            pl.BlockSpec((D, D), lambda i: (0, 0)),
            pl.BlockSpec((1, D), lambda i: (0, 0)),
        ],
        out_specs=pl.BlockSpec((BR, D), lambda i: (i, 0)),
        out_shape=jax.ShapeDtypeStruct((N_PAD, D), jnp.float32),
    )(agg, deg3, x, wn, ws, b.reshape(1, D))


def kernel(node_fts, edge_index, W_neigh1, W_self1, b1, W_neigh2, W_self2, b2):
    src = edge_index[0]
    dst = edge_index[1]
    pad = E_PAD - E
    src_p = jnp.concatenate([src, jnp.zeros((pad,), jnp.int32)]).reshape(NW, KB, B)
    dst_p = jnp.concatenate([dst, jnp.full((pad,), N, jnp.int32)]).reshape(NW, KB, B)
    x0 = jnp.pad(node_fts, ((0, N_PAD - N), (0, 0)))

    agg1, deg = _sc_agg_deg(x0, src_p, dst_p)
    deg3 = deg.reshape(NC, N_PAD, 1)
    out1 = _tc_layer(agg1, deg3, x0, W_neigh1, W_self1, b1)
    (agg2,) = _sc_agg(out1, src_p, dst_p)
    out2 = _tc_layer(agg2, deg3, out1, W_neigh2, W_self2, b2)
    return out2[:N]
